# baseline (speedup 1.0000x reference)
import functools

import jax
import jax.numpy as jnp
from jax import lax
from jax.experimental import pallas as pl
from jax.experimental.pallas import tpu as pltpu
from jax.experimental.pallas import tpu_sc as plsc

_N = 10000
_NP = 10240
_E = 320000
_EP = 327680
_IN = 128
_C = 16
_K = 10
_BANKS = 9
_COLS = 144
_PCOLS = 160
_HALF = 80
_G = _HALF // 16
_NS = 16
_RPT = _NP // _NS
_RSUB = 64
_NSUB = _RPT // _RSUB
_EPT = _EP // _NS
_ECH = 64
_NCH = _EPT // _ECH
_NPAIR = _NCH // 2
_GCH = _EP // _ECH
_EBLK = 16


def _matmul_body(x_ref, w_ref, b_ref, o_ref):
    o_ref[0] = (
        jnp.dot(x_ref[...], w_ref[0], preferred_element_type=jnp.float32)
        + b_ref[0]
    )


def _bcast_lane(v, e):
    idx = jnp.full((16, 1), e, dtype=jnp.int32)
    dn = lax.GatherDimensionNumbers(
        offset_dims=(), collapsed_slice_dims=(0,), start_index_map=(0,)
    )
    return lax.gather(
        v, idx, dn, (1,), mode=lax.GatherScatterMode.PROMISE_IN_BOUNDS
    )


def _sc_body(
    h_hbm, edata_hbm, at_hbm, bt_hbm, za_hbm,
    z_hbm, s0_hbm, s1_hbm,
    acc_sh, z_vm, ra_vm, rb_vm, src_vm, old_vm, ea_vm, eb_vm,
    a_vm, b_vm, za_vm, sem_a, sem_b, sem_ea, sem_eb,
):
    c = lax.axis_index("c")
    s = lax.axis_index("s")
    row0 = s * _RPT
    e0 = s * _EPT

    pltpu.sync_copy(at_hbm.at[c], a_vm)
    pltpu.sync_copy(bt_hbm.at[c], b_vm)
    pltpu.sync_copy(za_hbm.at[c], za_vm)

    def zero_body(r, _):
        for g in range(_G):
            z_vm[r, pl.ds(g * 16, 16)] = jnp.zeros((16,), jnp.float32)
        return 0

    lax.fori_loop(0, _RPT, zero_body, 0)

    for sub in range(_NSUB):
        r0 = row0 + sub * _RSUB
        pltpu.sync_copy(h_hbm.at[c, pl.ds(r0, _RSUB)], src_vm)
        pltpu.sync_copy(src_vm, s0_hbm.at[c, pl.ds(r0, _RSUB)])
        pltpu.sync_copy(src_vm, s1_hbm.at[c, pl.ds(r0, _RSUB)])

    def hop(k, src_ref, old_ref, dst_ref):
        for sub in range(_NSUB):
            r0 = row0 + sub * _RSUB
            pltpu.sync_copy(src_ref.at[c, pl.ds(r0, _RSUB)], src_vm)
            pltpu.sync_copy(old_ref.at[c, pl.ds(r0, _RSUB)], old_vm)

            def init_body(r, _, sub=sub, k=k):
                for g in range(_G):
                    sl = pl.ds(g * 16, 16)
                    sv = src_vm[r, sl]
                    acc = a_vm[k - 1, sl] * sv + b_vm[k - 1, sl] * old_vm[r, sl]
                    old_vm[r, sl] = acc
                    zr = sub * _RSUB + r
                    z_vm[zr, sl] = z_vm[zr, sl] + za_vm[k - 1, sl] * sv
                return 0

            lax.fori_loop(0, _RSUB, init_body, 0)
            pltpu.sync_copy(old_vm, acc_sh.at[pl.ds(r0, _RSUB)])
        plsc.subcore_barrier()

        def gstart(e_vm, j, r_vm, sem):
            pltpu.async_copy(src_ref.at[c].at[e_vm.at[j, 0]], r_vm, sem)

        def gwait(e_vm, j, r_vm, sem):
            pltpu.make_async_copy(
                src_ref.at[c].at[e_vm.at[j, 0]], r_vm, sem
            ).wait()

        def estart(e_vm, p, sem):
            pltpu.async_copy(edata_hbm.at[p], e_vm, sem)

        def ewait(e_vm, p, sem):
            pltpu.make_async_copy(edata_hbm.at[p], e_vm, sem).wait()

        def scale_scatter(e_vm, j, r_vm):
            def scale_body(bi, _):
                base = bi * _EBLK
                wv = plsc.bitcast(e_vm[j, 2, pl.ds(base, _EBLK)], jnp.float32)
                for e in range(_EBLK):
                    wsp = _bcast_lane(wv, e)
                    r = base + e
                    for g in range(_G):
                        sl = pl.ds(g * 16, 16)
                        r_vm[r, sl] = r_vm[r, sl] * wsp
                return 0

            lax.fori_loop(0, _ECH // _EBLK, scale_body, 0)
            pltpu.sync_copy(r_vm, acc_sh.at[e_vm.at[j, 1]], add=True)

        p0 = s * (_NCH // 2)
        pltpu.sync_copy(edata_hbm.at[p0], ea_vm)
        gstart(ea_vm, 0, ra_vm, sem_a)
        gstart(ea_vm, 1, rb_vm, sem_b)
        estart(eb_vm, p0 + 1, sem_eb)

        def quad_body(i, _):
            pa = p0 + 2 * i
            pb = pa + 1
            gwait(ea_vm, 0, ra_vm, sem_a)
            scale_scatter(ea_vm, 0, ra_vm)
            ewait(eb_vm, pb, sem_eb)
            gstart(eb_vm, 0, ra_vm, sem_a)
            gwait(ea_vm, 1, rb_vm, sem_b)
            scale_scatter(ea_vm, 1, rb_vm)
            estart(ea_vm, pa + 2, sem_ea)
            gstart(eb_vm, 1, rb_vm, sem_b)
            gwait(eb_vm, 0, ra_vm, sem_a)
            scale_scatter(eb_vm, 0, ra_vm)
            ewait(ea_vm, pa + 2, sem_ea)
            gstart(ea_vm, 0, ra_vm, sem_a)
            gwait(eb_vm, 1, rb_vm, sem_b)
            scale_scatter(eb_vm, 1, rb_vm)
            gstart(ea_vm, 1, rb_vm, sem_b)
            estart(eb_vm, pb + 2, sem_eb)
            return 0

        lax.fori_loop(0, _NCH // 4, quad_body, 0)
        gwait(ea_vm, 0, ra_vm, sem_a)
        gwait(ea_vm, 1, rb_vm, sem_b)
        ewait(eb_vm, p0 + _NCH // 2 + 1, sem_eb)
        plsc.subcore_barrier()

        pltpu.sync_copy(
            acc_sh.at[pl.ds(row0, _RPT)], dst_ref.at[c, pl.ds(row0, _RPT)]
        )

    def dhop(i, _):
        k = 1 + 2 * i
        hop(k, s0_hbm, s1_hbm, s1_hbm)
        hop(k + 1, s1_hbm, s0_hbm, s0_hbm)
        return 0

    lax.fori_loop(0, _K // 2, dhop, 0)

    for sub in range(_NSUB):
        r0 = row0 + sub * _RSUB
        pltpu.sync_copy(acc_sh.at[pl.ds(r0, _RSUB)], src_vm)

        def fin_body(r, _, sub=sub):
            for g in range(_G):
                sl = pl.ds(g * 16, 16)
                zr = sub * _RSUB + r
                z_vm[zr, sl] = z_vm[zr, sl] + za_vm[_K, sl] * src_vm[r, sl]
            return 0

        lax.fori_loop(0, _RSUB, fin_body, 0)
    pltpu.sync_copy(z_vm, z_hbm.at[c, pl.ds(row0, _RPT)])


_sc_kernel = functools.partial(
    pl.kernel,
    _sc_body,
    out_type=(
        jax.ShapeDtypeStruct((2, _NP, _HALF), jnp.float32),
        jax.ShapeDtypeStruct((2, _NP, _HALF), jnp.float32),
        jax.ShapeDtypeStruct((2, _NP, _HALF), jnp.float32),
    ),
    mesh=plsc.VectorSubcoreMesh(core_axis_name="c", subcore_axis_name="s"),
    compiler_params=pltpu.CompilerParams(
        use_tc_tiling_on_sc=False, needs_layout_passes=False
    ),
    scratch_types=[
        pltpu.VMEM_SHARED((_NP, _HALF), jnp.float32),
        pltpu.VMEM((_RPT, _HALF), jnp.float32),
        pltpu.VMEM((_ECH, _HALF), jnp.float32),
        pltpu.VMEM((_ECH, _HALF), jnp.float32),
        pltpu.VMEM((_RSUB, _HALF), jnp.float32),
        pltpu.VMEM((_RSUB, _HALF), jnp.float32),
        pltpu.VMEM((2, 3, _ECH), jnp.int32),
        pltpu.VMEM((2, 3, _ECH), jnp.int32),
        pltpu.VMEM((_K, _HALF), jnp.float32),
        pltpu.VMEM((_K, _HALF), jnp.float32),
        pltpu.VMEM((_K + 1, _HALF), jnp.float32),
        pltpu.SemaphoreType.DMA,
        pltpu.SemaphoreType.DMA,
        pltpu.SemaphoreType.DMA,
        pltpu.SemaphoreType.DMA,
    ],
)


def _pad_cols(t):
    rows = t.shape[0]
    t = jnp.pad(t, ((0, 0), (0, _PCOLS - _COLS)))
    return t.reshape(rows, 2, _HALF).transpose(1, 0, 2)


@jax.jit
def kernel(x, edge_index, edge_weight, W, bias, alpha, a, b):
    f32 = jnp.float32
    a = a.astype(f32)
    b = b.astype(f32)
    ab = a + b

    c0 = (a - b) / 2.0
    c1 = (a + b + 2.0) / 2.0
    theta_prev = c1
    s_k = c1
    A_rows = [c0 / c1]
    B_rows = [jnp.zeros_like(a)]
    ZA_rows = [alpha[:, 0, :].astype(f32)]
    ZA_rows.append(alpha[:, 1, :].astype(f32) * s_k[:, None])
    for k in range(2, _K + 1):
        th = (2 * k + ab) * (2 * k + ab - 1) / (2 * k * (k + ab))
        thp = (2 * k + ab - 1) * (a ** 2 - b ** 2) / (
            2 * k * (k + ab) * (2 * k + ab - 2)
        )
        thd = (k + a - 1) * (k + b - 1) * (2 * k + ab) / (
            k * (k + ab) * (2 * k + ab - 2)
        )
        A_rows.append(thp / th)
        B_rows.append(-thd / (th * theta_prev))
        s_k = th * s_k
        ZA_rows.append(alpha[:, k, :].astype(f32) * s_k[:, None])
        theta_prev = th

    A_tab = _pad_cols(jnp.stack([jnp.repeat(r, _C) for r in A_rows]))
    B_tab = _pad_cols(jnp.stack([jnp.repeat(r, _C) for r in B_rows]))
    ZA_tab = _pad_cols(jnp.stack([r.reshape(_COLS) for r in ZA_rows]))

    Wf = jnp.transpose(W.astype(f32), (1, 0, 2)).reshape(_IN, _COLS)
    Wp = jnp.pad(Wf, ((0, 0), (0, _PCOLS - _COLS)))
    Wp = Wp.reshape(_IN, 2, _HALF).transpose(1, 0, 2)
    bp = jnp.pad(bias.astype(f32).reshape(_COLS), (0, _PCOLS - _COLS))
    bp = bp.reshape(2, 1, _HALF)
    xp = jnp.pad(x.astype(f32), ((0, _NP - _N), (0, 0)))
    h2 = pl.pallas_call(
        _matmul_body,
        grid=(2,),
        in_specs=[
            pl.BlockSpec((_NP, _IN), lambda g: (0, 0)),
            pl.BlockSpec((1, _IN, _HALF), lambda g: (g, 0, 0)),
            pl.BlockSpec((1, 1, _HALF), lambda g: (g, 0, 0)),
        ],
        out_specs=pl.BlockSpec((1, _NP, _HALF), lambda g: (g, 0, 0)),
        out_shape=jax.ShapeDtypeStruct((2, _NP, _HALF), f32),
    )(xp, Wp, bp)

    order = jnp.argsort(edge_index[1])
    cidx = jnp.pad(edge_index[1][order].astype(jnp.int32), (0, _EP - _E))
    ridx = jnp.pad(edge_index[0][order].astype(jnp.int32), (0, _EP - _E))
    ew = jnp.pad(edge_weight[order].astype(f32), (0, _EP - _E))
    wbits = lax.bitcast_convert_type(ew, jnp.int32)
    edata = jnp.stack(
        [cidx.reshape(_GCH, _ECH), ridx.reshape(_GCH, _ECH),
         wbits.reshape(_GCH, _ECH)],
        axis=1,
    ).reshape(_GCH // 2, 2, 3, _ECH)
    edata = jnp.pad(edata, ((0, 2), (0, 0), (0, 0), (0, 0)))
    z2, _, _ = _sc_kernel()(h2, edata, A_tab, B_tab, ZA_tab)

    z = z2[:, :_N, :].transpose(1, 0, 2).reshape(_N, _PCOLS)[:, :_COLS]
    return z.reshape(_N, _BANKS, _C).transpose(1, 0, 2)


# speedup vs baseline: 2.3327x; 2.3327x over previous
import functools

import jax
import jax.numpy as jnp
from jax import lax
from jax.experimental import pallas as pl
from jax.experimental.pallas import tpu as pltpu
from jax.experimental.pallas import tpu_sc as plsc

_N = 10000
_NP = 10240
_E = 320000
_EP = 327680
_IN = 128
_C = 16
_K = 10
_BANKS = 9
_COLS = 144
_PCOLS = 160
_HALF = 80
_G = _HALF // 16
_NS = 16
_RPT = _NP // _NS
_RSUB = 64
_NSUB = _RPT // _RSUB
_EPT = _EP // _NS
_ECH = 64
_NCH = _EPT // _ECH
_NPAIR = _NCH // 2
_GCH = _EP // _ECH
_EBLK = 16


def _matmul_body(x_ref, w_ref, b_ref, o_ref):
    o_ref[0] = (
        jnp.dot(x_ref[...], w_ref[0], preferred_element_type=jnp.float32)
        + b_ref[0]
    )


def _bcast_lane(v, e):
    idx = jnp.full((16, 1), e, dtype=jnp.int32)
    dn = lax.GatherDimensionNumbers(
        offset_dims=(), collapsed_slice_dims=(0,), start_index_map=(0,)
    )
    return lax.gather(
        v, idx, dn, (1,), mode=lax.GatherScatterMode.PROMISE_IN_BOUNDS
    )


def _sc_body(
    h_hbm, edata_hbm, at_hbm, bt_hbm, za_hbm,
    z_hbm,
    u_sh, v_sh, ra_vm, rb_vm, zc_vm, ea_vm, eb_vm,
    a_vm, b_vm, za_vm, sem_a, sem_b, sem_ea, sem_eb,
):
    c = lax.axis_index("c")
    s = lax.axis_index("s")
    row0 = s * _RPT

    pltpu.sync_copy(at_hbm.at[c], a_vm)
    pltpu.sync_copy(bt_hbm.at[c], b_vm)
    pltpu.sync_copy(za_hbm.at[c], za_vm)

    def zero_body(r, _):
        for g in range(_G):
            zc_vm[r, pl.ds(g * 16, 16)] = jnp.zeros((16,), jnp.float32)
        return 0

    lax.fori_loop(0, _RSUB, zero_body, 0)
    for sub in range(_NSUB):
        r0 = row0 + sub * _RSUB
        pltpu.sync_copy(zc_vm, z_hbm.at[c, pl.ds(r0, _RSUB)])

    pltpu.sync_copy(h_hbm.at[c, pl.ds(row0, _RPT)], u_sh.at[pl.ds(row0, _RPT)])
    pltpu.sync_copy(h_hbm.at[c, pl.ds(row0, _RPT)], v_sh.at[pl.ds(row0, _RPT)])

    def hop(k, src_sh, acc_sh):
        for sub in range(_NSUB):
            r0 = row0 + sub * _RSUB
            pltpu.sync_copy(src_sh.at[pl.ds(r0, _RSUB)], ra_vm)
            pltpu.sync_copy(acc_sh.at[pl.ds(r0, _RSUB)], rb_vm)
            pltpu.sync_copy(z_hbm.at[c, pl.ds(r0, _RSUB)], zc_vm)

            def init_body(r, _, k=k):
                for g in range(_G):
                    sl = pl.ds(g * 16, 16)
                    sv = ra_vm[r, sl]
                    rb_vm[r, sl] = (
                        a_vm[k - 1, sl] * sv + b_vm[k - 1, sl] * rb_vm[r, sl]
                    )
                    zc_vm[r, sl] = zc_vm[r, sl] + za_vm[k - 1, sl] * sv
                return 0

            lax.fori_loop(0, _RSUB, init_body, 0)
            pltpu.sync_copy(rb_vm, acc_sh.at[pl.ds(r0, _RSUB)])
            pltpu.sync_copy(zc_vm, z_hbm.at[c, pl.ds(r0, _RSUB)])
        plsc.subcore_barrier()

        def gstart(e_vm, j, r_vm, sem):
            pltpu.async_copy(src_sh.at[e_vm.at[j, 0]], r_vm, sem)

        def gwait(e_vm, j, r_vm, sem):
            pltpu.make_async_copy(src_sh.at[e_vm.at[j, 0]], r_vm, sem).wait()

        def estart(e_vm, p, sem):
            pltpu.async_copy(edata_hbm.at[p], e_vm, sem)

        def ewait(e_vm, p, sem):
            pltpu.make_async_copy(edata_hbm.at[p], e_vm, sem).wait()

        def scale_scatter(e_vm, j, r_vm):
            def scale_body(bi, _):
                base = bi * _EBLK
                wv = plsc.bitcast(e_vm[j, 2, pl.ds(base, _EBLK)], jnp.float32)
                for e in range(_EBLK):
                    wsp = _bcast_lane(wv, e)
                    r = base + e
                    for g in range(_G):
                        sl = pl.ds(g * 16, 16)
                        r_vm[r, sl] = r_vm[r, sl] * wsp
                return 0

            lax.fori_loop(0, _ECH // _EBLK, scale_body, 0)
            pltpu.sync_copy(r_vm, acc_sh.at[e_vm.at[j, 1]], add=True)

        p0 = s * (_NCH // 2)
        pltpu.sync_copy(edata_hbm.at[p0], ea_vm)
        gstart(ea_vm, 0, ra_vm, sem_a)
        gstart(ea_vm, 1, rb_vm, sem_b)
        estart(eb_vm, p0 + 1, sem_eb)

        def quad_body(i, _):
            pa = p0 + 2 * i
            pb = pa + 1
            gwait(ea_vm, 0, ra_vm, sem_a)
            scale_scatter(ea_vm, 0, ra_vm)
            ewait(eb_vm, pb, sem_eb)
            gstart(eb_vm, 0, ra_vm, sem_a)
            gwait(ea_vm, 1, rb_vm, sem_b)
            scale_scatter(ea_vm, 1, rb_vm)
            estart(ea_vm, pa + 2, sem_ea)
            gstart(eb_vm, 1, rb_vm, sem_b)
            gwait(eb_vm, 0, ra_vm, sem_a)
            scale_scatter(eb_vm, 0, ra_vm)
            ewait(ea_vm, pa + 2, sem_ea)
            gstart(ea_vm, 0, ra_vm, sem_a)
            gwait(eb_vm, 1, rb_vm, sem_b)
            scale_scatter(eb_vm, 1, rb_vm)
            gstart(ea_vm, 1, rb_vm, sem_b)
            estart(eb_vm, pb + 2, sem_eb)
            return 0

        lax.fori_loop(0, _NCH // 4, quad_body, 0)
        gwait(ea_vm, 0, ra_vm, sem_a)
        gwait(ea_vm, 1, rb_vm, sem_b)
        ewait(eb_vm, p0 + _NCH // 2 + 1, sem_eb)
        plsc.subcore_barrier()

    def dhop(i, _):
        k = 1 + 2 * i
        hop(k, u_sh, v_sh)
        hop(k + 1, v_sh, u_sh)
        return 0

    lax.fori_loop(0, _K // 2, dhop, 0)

    for sub in range(_NSUB):
        r0 = row0 + sub * _RSUB
        pltpu.sync_copy(u_sh.at[pl.ds(r0, _RSUB)], ra_vm)
        pltpu.sync_copy(z_hbm.at[c, pl.ds(r0, _RSUB)], zc_vm)

        def fin_body(r, _):
            for g in range(_G):
                sl = pl.ds(g * 16, 16)
                zc_vm[r, sl] = zc_vm[r, sl] + za_vm[_K, sl] * ra_vm[r, sl]
            return 0

        lax.fori_loop(0, _RSUB, fin_body, 0)
        pltpu.sync_copy(zc_vm, z_hbm.at[c, pl.ds(r0, _RSUB)])


_sc_kernel = functools.partial(
    pl.kernel,
    _sc_body,
    out_type=jax.ShapeDtypeStruct((2, _NP, _HALF), jnp.float32),
    mesh=plsc.VectorSubcoreMesh(core_axis_name="c", subcore_axis_name="s"),
    compiler_params=pltpu.CompilerParams(
        use_tc_tiling_on_sc=False, needs_layout_passes=False
    ),
    scratch_types=[
        pltpu.VMEM_SHARED((_NP, _HALF), jnp.float32),
        pltpu.VMEM_SHARED((_NP, _HALF), jnp.float32),
        pltpu.VMEM((_RSUB, _HALF), jnp.float32),
        pltpu.VMEM((_RSUB, _HALF), jnp.float32),
        pltpu.VMEM((_RSUB, _HALF), jnp.float32),
        pltpu.VMEM((2, 3, _ECH), jnp.int32),
        pltpu.VMEM((2, 3, _ECH), jnp.int32),
        pltpu.VMEM((_K, _HALF), jnp.float32),
        pltpu.VMEM((_K, _HALF), jnp.float32),
        pltpu.VMEM((_K + 1, _HALF), jnp.float32),
        pltpu.SemaphoreType.DMA,
        pltpu.SemaphoreType.DMA,
        pltpu.SemaphoreType.DMA,
        pltpu.SemaphoreType.DMA,
    ],
)


def _pad_cols(t):
    rows = t.shape[0]
    t = jnp.pad(t, ((0, 0), (0, _PCOLS - _COLS)))
    return t.reshape(rows, 2, _HALF).transpose(1, 0, 2)


@jax.jit
def kernel(x, edge_index, edge_weight, W, bias, alpha, a, b):
    f32 = jnp.float32
    a = a.astype(f32)
    b = b.astype(f32)
    ab = a + b

    c0 = (a - b) / 2.0
    c1 = (a + b + 2.0) / 2.0
    theta_prev = c1
    s_k = c1
    A_rows = [c0 / c1]
    B_rows = [jnp.zeros_like(a)]
    ZA_rows = [alpha[:, 0, :].astype(f32)]
    ZA_rows.append(alpha[:, 1, :].astype(f32) * s_k[:, None])
    for k in range(2, _K + 1):
        th = (2 * k + ab) * (2 * k + ab - 1) / (2 * k * (k + ab))
        thp = (2 * k + ab - 1) * (a ** 2 - b ** 2) / (
            2 * k * (k + ab) * (2 * k + ab - 2)
        )
        thd = (k + a - 1) * (k + b - 1) * (2 * k + ab) / (
            k * (k + ab) * (2 * k + ab - 2)
        )
        A_rows.append(thp / th)
        B_rows.append(-thd / (th * theta_prev))
        s_k = th * s_k
        ZA_rows.append(alpha[:, k, :].astype(f32) * s_k[:, None])
        theta_prev = th

    A_tab = _pad_cols(jnp.stack([jnp.repeat(r, _C) for r in A_rows]))
    B_tab = _pad_cols(jnp.stack([jnp.repeat(r, _C) for r in B_rows]))
    ZA_tab = _pad_cols(jnp.stack([r.reshape(_COLS) for r in ZA_rows]))

    Wf = jnp.transpose(W.astype(f32), (1, 0, 2)).reshape(_IN, _COLS)
    Wp = jnp.pad(Wf, ((0, 0), (0, _PCOLS - _COLS)))
    Wp = Wp.reshape(_IN, 2, _HALF).transpose(1, 0, 2)
    bp = jnp.pad(bias.astype(f32).reshape(_COLS), (0, _PCOLS - _COLS))
    bp = bp.reshape(2, 1, _HALF)
    xp = jnp.pad(x.astype(f32), ((0, _NP - _N), (0, 0)))
    h2 = pl.pallas_call(
        _matmul_body,
        grid=(2,),
        in_specs=[
            pl.BlockSpec((_NP, _IN), lambda g: (0, 0)),
            pl.BlockSpec((1, _IN, _HALF), lambda g: (g, 0, 0)),
            pl.BlockSpec((1, 1, _HALF), lambda g: (g, 0, 0)),
        ],
        out_specs=pl.BlockSpec((1, _NP, _HALF), lambda g: (g, 0, 0)),
        out_shape=jax.ShapeDtypeStruct((2, _NP, _HALF), f32),
    )(xp, Wp, bp)

    cidx = jnp.pad(edge_index[1].astype(jnp.int32), (0, _EP - _E))
    ridx = jnp.pad(edge_index[0].astype(jnp.int32), (0, _EP - _E))
    ew = jnp.pad(edge_weight.astype(f32), (0, _EP - _E))
    wbits = lax.bitcast_convert_type(ew, jnp.int32)
    edata = jnp.stack(
        [cidx.reshape(_GCH, _ECH), ridx.reshape(_GCH, _ECH),
         wbits.reshape(_GCH, _ECH)],
        axis=1,
    ).reshape(_GCH // 2, 2, 3, _ECH)
    edata = jnp.pad(edata, ((0, 2), (0, 0), (0, 0), (0, 0)))
    z2 = _sc_kernel()(h2, edata, A_tab, B_tab, ZA_tab)

    z = z2[:, :_N, :].transpose(1, 0, 2).reshape(_N, _PCOLS)[:, :_COLS]
    return z.reshape(_N, _BANKS, _C).transpose(1, 0, 2)
